# baseline (speedup 1.0000x reference)
import functools

import jax
import jax.numpy as jnp
from jax import lax
from jax.experimental import pallas as pl
from jax.experimental.pallas import tpu as pltpu
from jax.experimental.pallas import tpu_sc as plsc

B = 8
IN = 16
OUT = 8
NE = 8192
NCPS = 64
TABW = 64

NROWS = B * IN
NWORKERS = 32
ROWS_PER_W = NROWS // NWORKERS
LANES = 16

_mesh = plsc.VectorSubcoreMesh(core_axis_name="c", subcore_axis_name="s")

CH = 4096
NCHUNK = ROWS_PER_W * (NE // CH)


@functools.partial(
    pl.kernel,
    out_type=jax.ShapeDtypeStruct((NROWS, OUT, NE), jnp.float32),
    mesh=_mesh,
    scratch_types=[
        pltpu.VMEM((IN * OUT, TABW), jnp.int32),
        pltpu.VMEM((IN * OUT, TABW), jnp.int32),
        pltpu.VMEM((2, CH), jnp.float32),
        pltpu.VMEM((2, OUT, CH), jnp.float32),
        pltpu.SemaphoreType.DMA,
        pltpu.SemaphoreType.DMA,
        pltpu.SemaphoreType.DMA,
        pltpu.SemaphoreType.DMA,
    ],
    compiler_params=pltpu.CompilerParams(needs_layout_passes=False, skip_device_barrier=True),
)
def _bspline_sc(x_hbm, ta_hbm, tb_hbm, out_hbm, tA, tB, xbuf, obuf,
                sx0, sx1, so0, so1):
    wid = lax.axis_index("s") * 2 + lax.axis_index("c")
    sem_x = (sx0, sx1)
    sem_o = (so0, so1)
    pltpu.sync_copy(ta_hbm, tA)
    pltpu.sync_copy(tb_hbm, tB)

    def x_src(t):
        row = wid * ROWS_PER_W + t // 2
        return x_hbm.at[row, pl.ds((t % 2) * CH, CH)]

    xh = [None, None]
    oh = [None, None]
    xh[0] = pltpu.async_copy(x_src(0), xbuf.at[0], sem_x[0])

    for t in range(NCHUNK):
        p = t & 1
        row = wid * ROWS_PER_W + t // 2
        i_idx = lax.rem(row, IN)
        iob = i_idx * OUT

        xh[p].wait()
        if t + 1 < NCHUNK:
            xh[1 - p] = pltpu.async_copy(x_src(t + 1), xbuf.at[1 - p], sem_x[1 - p])
        if t >= 2:
            oh[p].wait()

        @plsc.parallel_loop(0, CH, LANES, unroll=2)
        def _loop(e0):
            xv = xbuf[p, pl.ds(e0, LANES)]
            xi = xv * 62.0
            ci = xi.astype(jnp.int32)
            u = xi - ci.astype(jnp.float32)
            for o in range(OUT):
                wa = plsc.load_gather(tA.at[iob + o], [ci])
                wb = plsc.load_gather(tB.at[iob + o], [ci])
                a0 = plsc.bitcast(lax.shift_left(wa, 16), jnp.float32)
                a1 = plsc.bitcast(wa, jnp.float32)
                a2 = plsc.bitcast(lax.shift_left(wb, 16), jnp.float32)
                a3 = plsc.bitcast(wb, jnp.float32)
                res = ((a3 * u + a2) * u + a1) * u + a0
                obuf[p, o, pl.ds(e0, LANES)] = res

        oh[p] = pltpu.async_copy(
            obuf.at[p], out_hbm.at[row, :, pl.ds((t % 2) * CH, CH)], sem_o[p])

    oh[0].wait()
    oh[1].wait()


def _bf16_bits(a):
    b = lax.bitcast_convert_type(a.astype(jnp.bfloat16), jnp.uint16)
    return b.astype(jnp.uint32)


def kernel(x, cp):
    xf = x.reshape(NROWS, NE)
    edge = jnp.broadcast_to(cp[..., -1:], (IN, OUT, 3))
    cpe = jnp.concatenate([cp, edge], axis=-1)
    p0 = cpe[..., 0:TABW]
    p1 = cpe[..., 1:TABW + 1]
    p2 = cpe[..., 2:TABW + 2]
    p3 = cpe[..., 3:TABW + 3]
    a0 = (p0 + 4.0 * p1 + p2) * (1.0 / 6.0)
    a1 = (p2 - p0) * 0.5
    a2 = (p0 - 2.0 * p1 + p2) * 0.5
    a3 = (p3 - p0) * (1.0 / 6.0) + (p1 - p2) * 0.5
    ta = (_bf16_bits(a0) | (_bf16_bits(a1) << 16)).astype(jnp.int32)
    tb = (_bf16_bits(a2) | (_bf16_bits(a3) << 16)).astype(jnp.int32)
    out = _bspline_sc(xf, ta.reshape(IN * OUT, TABW), tb.reshape(IN * OUT, TABW))
    return out.reshape(B, IN, OUT, NE)


# speedup vs baseline: 1.0025x; 1.0025x over previous
import functools

import jax
import jax.numpy as jnp
from jax import lax
from jax.experimental import pallas as pl
from jax.experimental.pallas import tpu as pltpu
from jax.experimental.pallas import tpu_sc as plsc

B = 8
IN = 16
OUT = 8
NE = 8192
NCPS = 64
TABW = 64

NROWS = B * IN
NWORKERS = 32
ROWS_PER_W = NROWS // NWORKERS
LANES = 16

_mesh = plsc.VectorSubcoreMesh(core_axis_name="c", subcore_axis_name="s")

CH = 4096
NCHUNK = ROWS_PER_W * (NE // CH)


@functools.partial(
    pl.kernel,
    out_type=jax.ShapeDtypeStruct((NROWS, OUT, NE), jnp.float32),
    mesh=_mesh,
    scratch_types=[
        pltpu.VMEM((IN * OUT, TABW), jnp.int32),
        pltpu.VMEM((IN * OUT, TABW), jnp.int32),
        pltpu.VMEM((2, CH), jnp.float32),
        pltpu.VMEM((2, OUT, CH), jnp.float32),
        pltpu.SemaphoreType.DMA,
        pltpu.SemaphoreType.DMA,
        pltpu.SemaphoreType.DMA,
        pltpu.SemaphoreType.DMA,
    ],
    compiler_params=pltpu.CompilerParams(needs_layout_passes=False),
)
def _bspline_sc(x_hbm, ta_hbm, tb_hbm, out_hbm, tA, tB, xbuf, obuf,
                sx0, sx1, so0, so1):
    wid = lax.axis_index("s") * 2 + lax.axis_index("c")
    sem_x = (sx0, sx1)
    sem_o = (so0, so1)
    pltpu.sync_copy(ta_hbm, tA)
    pltpu.sync_copy(tb_hbm, tB)

    def x_src(t):
        row = wid * ROWS_PER_W + t // 2
        return x_hbm.at[row, pl.ds((t % 2) * CH, CH)]

    xh = [None, None]
    oh = [None, None]
    xh[0] = pltpu.async_copy(x_src(0), xbuf.at[0], sem_x[0])

    for t in range(NCHUNK):
        p = t & 1
        row = wid * ROWS_PER_W + t // 2
        i_idx = lax.rem(row, IN)
        iob = i_idx * OUT

        xh[p].wait()
        if t + 1 < NCHUNK:
            xh[1 - p] = pltpu.async_copy(x_src(t + 1), xbuf.at[1 - p], sem_x[1 - p])
        if t >= 2:
            oh[p].wait()

        @plsc.parallel_loop(0, CH, LANES, unroll=2)
        def _loop(e0):
            xv = xbuf[p, pl.ds(e0, LANES)]
            xi = xv * 62.0
            ci = xi.astype(jnp.int32)
            u = xi - ci.astype(jnp.float32)
            for o in range(OUT):
                wa = plsc.load_gather(tA.at[iob + o], [ci])
                wb = plsc.load_gather(tB.at[iob + o], [ci])
                a0 = plsc.bitcast(lax.shift_left(wa, 16), jnp.float32)
                a1 = plsc.bitcast(wa, jnp.float32)
                a2 = plsc.bitcast(lax.shift_left(wb, 16), jnp.float32)
                a3 = plsc.bitcast(wb, jnp.float32)
                res = ((a3 * u + a2) * u + a1) * u + a0
                obuf[p, o, pl.ds(e0, LANES)] = res

        oh[p] = pltpu.async_copy(
            obuf.at[p], out_hbm.at[row, :, pl.ds((t % 2) * CH, CH)], sem_o[p])

    oh[0].wait()
    oh[1].wait()


def _bf16_bits(a):
    b = lax.bitcast_convert_type(a.astype(jnp.bfloat16), jnp.uint16)
    return b.astype(jnp.uint32)


def kernel(x, cp):
    xf = x.reshape(NROWS, NE)
    edge = jnp.broadcast_to(cp[..., -1:], (IN, OUT, 3))
    cpe = jnp.concatenate([cp, edge], axis=-1)
    p0 = cpe[..., 0:TABW]
    p1 = cpe[..., 1:TABW + 1]
    p2 = cpe[..., 2:TABW + 2]
    p3 = cpe[..., 3:TABW + 3]
    a0 = (p0 + 4.0 * p1 + p2) * (1.0 / 6.0)
    a1 = (p2 - p0) * 0.5
    a2 = (p0 - 2.0 * p1 + p2) * 0.5
    a3 = (p3 - p0) * (1.0 / 6.0) + (p1 - p2) * 0.5
    ta = (_bf16_bits(a0) | (_bf16_bits(a1) << 16)).astype(jnp.int32)
    tb = (_bf16_bits(a2) | (_bf16_bits(a3) << 16)).astype(jnp.int32)
    out = _bspline_sc(xf, ta.reshape(IN * OUT, TABW), tb.reshape(IN * OUT, TABW))
    return out.reshape(B, IN, OUT, NE)


# speedup vs baseline: 1.0213x; 1.0188x over previous
import functools

import jax
import jax.numpy as jnp
from jax import lax
from jax.experimental import pallas as pl
from jax.experimental.pallas import tpu as pltpu
from jax.experimental.pallas import tpu_sc as plsc

B = 8
IN = 16
OUT = 8
NE = 8192
NCPS = 64
TABW = 64

NROWS = B * IN
NWORKERS = 32
ROWS_PER_W = NROWS // NWORKERS
LANES = 16

_mesh = plsc.VectorSubcoreMesh(core_axis_name="c", subcore_axis_name="s")

CH = 4096
NCHUNK = ROWS_PER_W * (NE // CH)


@functools.partial(
    pl.kernel,
    out_type=jax.ShapeDtypeStruct((NROWS, OUT, NE), jnp.float32),
    mesh=_mesh,
    scratch_types=[
        pltpu.VMEM((IN * OUT, TABW), jnp.int32),
        pltpu.VMEM((IN * OUT, TABW), jnp.int32),
        pltpu.VMEM((2, CH), jnp.float32),
        pltpu.VMEM((2, OUT, CH), jnp.float32),
        pltpu.SemaphoreType.DMA,
        pltpu.SemaphoreType.DMA,
        pltpu.SemaphoreType.DMA,
        pltpu.SemaphoreType.DMA,
        pltpu.SemaphoreType.DMA,
        pltpu.SemaphoreType.DMA,
    ],
    compiler_params=pltpu.CompilerParams(needs_layout_passes=False),
)
def _bspline_sc(x_hbm, ta_hbm, tb_hbm, out_hbm, tA, tB, xbuf, obuf,
                sx0, sx1, so0, so1, sta, stb):
    wid = lax.axis_index("s") * 2 + lax.axis_index("c")
    sem_x = (sx0, sx1)
    sem_o = (so0, so1)

    def x_src(t):
        row = wid * ROWS_PER_W + t // 2
        return x_hbm.at[row, pl.ds((t % 2) * CH, CH)]

    xh = [None, None]
    oh = [None, None]
    xh[0] = pltpu.async_copy(x_src(0), xbuf.at[0], sem_x[0])
    tha = pltpu.async_copy(ta_hbm, tA, sta)
    thb = pltpu.async_copy(tb_hbm, tB, stb)

    for t in range(NCHUNK):
        if t == 0:
            tha.wait()
            thb.wait()
        p = t & 1
        row = wid * ROWS_PER_W + t // 2
        i_idx = lax.rem(row, IN)
        iob = i_idx * OUT

        xh[p].wait()
        if t + 1 < NCHUNK:
            xh[1 - p] = pltpu.async_copy(x_src(t + 1), xbuf.at[1 - p], sem_x[1 - p])
        if t >= 2:
            oh[p].wait()

        @plsc.parallel_loop(0, CH, LANES, unroll=2)
        def _loop(e0):
            xv = xbuf[p, pl.ds(e0, LANES)]
            xi = xv * 62.0
            ci = xi.astype(jnp.int32)
            u = xi - ci.astype(jnp.float32)
            for o in range(OUT):
                wa = plsc.load_gather(tA.at[iob + o], [ci])
                wb = plsc.load_gather(tB.at[iob + o], [ci])
                a0 = plsc.bitcast(lax.shift_left(wa, 16), jnp.float32)
                a1 = plsc.bitcast(wa, jnp.float32)
                a2 = plsc.bitcast(lax.shift_left(wb, 16), jnp.float32)
                a3 = plsc.bitcast(wb, jnp.float32)
                res = ((a3 * u + a2) * u + a1) * u + a0
                obuf[p, o, pl.ds(e0, LANES)] = res

        oh[p] = pltpu.async_copy(
            obuf.at[p], out_hbm.at[row, :, pl.ds((t % 2) * CH, CH)], sem_o[p])

    oh[0].wait()
    oh[1].wait()


def _bf16_bits(a):
    b = lax.bitcast_convert_type(a.astype(jnp.bfloat16), jnp.uint16)
    return b.astype(jnp.uint32)


def kernel(x, cp):
    xf = x.reshape(NROWS, NE)
    edge = jnp.broadcast_to(cp[..., -1:], (IN, OUT, 3))
    cpe = jnp.concatenate([cp, edge], axis=-1)
    p0 = cpe[..., 0:TABW]
    p1 = cpe[..., 1:TABW + 1]
    p2 = cpe[..., 2:TABW + 2]
    p3 = cpe[..., 3:TABW + 3]
    a0 = (p0 + 4.0 * p1 + p2) * (1.0 / 6.0)
    a1 = (p2 - p0) * 0.5
    a2 = (p0 - 2.0 * p1 + p2) * 0.5
    a3 = (p3 - p0) * (1.0 / 6.0) + (p1 - p2) * 0.5
    ta = (_bf16_bits(a0) | (_bf16_bits(a1) << 16)).astype(jnp.int32)
    tb = (_bf16_bits(a2) | (_bf16_bits(a3) << 16)).astype(jnp.int32)
    out = _bspline_sc(xf, ta.reshape(IN * OUT, TABW), tb.reshape(IN * OUT, TABW))
    return out.reshape(B, IN, OUT, NE)
